# baseline (speedup 1.0000x reference)
import functools

import jax
import jax.numpy as jnp
from jax import lax
from jax.experimental import pallas as pl
from jax.experimental.pallas import tpu as pltpu
from jax.experimental.pallas import tpu_sc as plsc

_TEMP_INV = 10.0
_LOG2E = 1.4426950408889634
_B = 1024
_D = 32
_M = 100000
_TM = 10000

_SC_INFO = plsc.get_sparse_core_info()
_NW = _SC_INFO.num_cores * _SC_INFO.num_subcores
_BPW = _B // _NW


def _sc_gather_body(idx_hbm, table_hbm, out_hbm, idx_v, rows_v, sem):
    wid = lax.axis_index("s") * _SC_INFO.num_cores + lax.axis_index("c")
    base = wid * _BPW
    pltpu.sync_copy(idx_hbm.at[pl.ds(base, _BPW)], idx_v)
    pltpu.async_copy(table_hbm.at[idx_v], rows_v, sem).wait()
    pltpu.sync_copy(rows_v, out_hbm.at[pl.ds(base, _BPW)])


_sc_gather = functools.partial(
    pl.kernel,
    mesh=plsc.VectorSubcoreMesh(core_axis_name="c", subcore_axis_name="s"),
    compiler_params=pltpu.CompilerParams(use_tc_tiling_on_sc=True),
    out_type=jax.ShapeDtypeStruct((_B, 4 * _D), jnp.float32),
    scratch_types=[
        pltpu.VMEM((_BPW,), jnp.int32),
        pltpu.VMEM((_BPW, 4 * _D), jnp.float32),
        pltpu.SemaphoreType.DMA,
    ],
)(_sc_gather_body)


def _fused_kernel(ft_ref, c_ref, lrowst_ref, out_ref, ft_scr, down_ref):
    i = pl.program_id(0)
    nt = pl.num_programs(0)

    @pl.when(i == 0)
    def _init():
        ft = ft_ref[...]
        nrm = jnp.sqrt(jnp.sum(ft * ft, axis=0, keepdims=True))
        ft_scr[...] = ft / jnp.maximum(nrm, 1e-12)
        down_ref[...] = jnp.zeros_like(down_ref)

    ft = ft_scr[...]
    c = c_ref[...]
    g = jax.lax.dot_general(
        c, ft, (((1,), (0,)), ((), ())), preferred_element_type=jnp.float32
    )
    e = jnp.exp2(g * (_TEMP_INV * _LOG2E))
    down_ref[...] += jnp.sum(e, axis=0, keepdims=True)

    @pl.when(i == nt - 1)
    def _fin():
        fb = ft.astype(jnp.bfloat16).astype(jnp.float32)
        rb = lrowst_ref[...].astype(jnp.bfloat16).astype(jnp.float32)
        gl = jnp.sum(fb * rb, axis=0, keepdims=True)
        ups = jnp.exp2(gl * (_TEMP_INV * _LOG2E))
        out_ref[...] = ups / down_ref[...]


@functools.partial(jax.jit, static_argnames=())
def kernel(features, labels, centrals):
    ft = features.T
    rows4 = _sc_gather(labels >> 2, centrals.reshape(_M // 4, 4 * _D))
    sub = rows4.reshape(_B, 4, _D)
    sel = (labels & 3)[:, None]
    lrows = jnp.where(
        sel == 0,
        sub[:, 0],
        jnp.where(sel == 1, sub[:, 1], jnp.where(sel == 2, sub[:, 2], sub[:, 3])),
    )
    lrowst = lrows.T
    nt = _M // _TM
    out = pl.pallas_call(
        _fused_kernel,
        grid=(nt,),
        in_specs=[
            pl.BlockSpec((_D, _B), lambda i: (0, 0)),
            pl.BlockSpec((_TM, _D), lambda i: (i, 0)),
            pl.BlockSpec((_D, _B), lambda i: (0, 0)),
        ],
        out_specs=pl.BlockSpec((1, _B), lambda i: (0, 0)),
        out_shape=jax.ShapeDtypeStruct((1, _B), jnp.float32),
        scratch_shapes=[
            pltpu.VMEM((_D, _B), jnp.float32),
            pltpu.VMEM((1, _B), jnp.float32),
        ],
    )(ft, centrals, lrowst)
    return out.reshape(_B)


# speedup vs baseline: 1.0716x; 1.0716x over previous
import functools

import jax
import jax.numpy as jnp
from jax import lax
from jax.experimental import pallas as pl
from jax.experimental.pallas import tpu as pltpu
from jax.experimental.pallas import tpu_sc as plsc

_TEMP_INV = 10.0
_LOG2E = 1.4426950408889634
_B = 1024
_D = 32
_M = 100000
_TMB = 1000

_SC_INFO = plsc.get_sparse_core_info()
_NW = _SC_INFO.num_cores * _SC_INFO.num_subcores
_BPW = _B // _NW


def _sc_gather_body(idx_hbm, table_hbm, out_hbm, idx_v, rows_v, sem):
    wid = lax.axis_index("s") * _SC_INFO.num_cores + lax.axis_index("c")
    base = wid * _BPW
    pltpu.sync_copy(idx_hbm.at[pl.ds(base, _BPW)], idx_v)
    pltpu.async_copy(table_hbm.at[idx_v], rows_v, sem).wait()
    pltpu.sync_copy(rows_v, out_hbm.at[pl.ds(base, _BPW)])


_sc_gather = functools.partial(
    pl.kernel,
    mesh=plsc.VectorSubcoreMesh(core_axis_name="c", subcore_axis_name="s"),
    compiler_params=pltpu.CompilerParams(use_tc_tiling_on_sc=True),
    out_type=jax.ShapeDtypeStruct((_B, 4 * _D), jnp.float32),
    scratch_types=[
        pltpu.VMEM((_BPW,), jnp.int32),
        pltpu.VMEM((_BPW, 4 * _D), jnp.float32),
        pltpu.SemaphoreType.DMA,
    ],
)(_sc_gather_body)


def _fused_kernel(ft_ref, c4_ref, lrowst_ref, out_ref, ft_scr, fblk_scr, down_ref):
    i = pl.program_id(0)
    nt = pl.num_programs(0)

    @pl.when(i == 0)
    def _init():
        ft = ft_ref[...]
        nrm = jnp.sqrt(jnp.sum(ft * ft, axis=0, keepdims=True))
        ftn = ft / jnp.maximum(nrm, 1e-12)
        ft_scr[...] = ftn
        fblk_scr[...] = jnp.zeros_like(fblk_scr)
        for q in range(4):
            fblk_scr[q * _D:(q + 1) * _D, q * _B:(q + 1) * _B] = ftn
        down_ref[...] = jnp.zeros_like(down_ref)

    fblk = fblk_scr[...]
    c4 = c4_ref[...]
    gg = jax.lax.dot_general(
        c4, fblk, (((1,), (0,)), ((), ())), preferred_element_type=jnp.float32
    )
    e = jnp.exp2(gg * (_TEMP_INV * _LOG2E))
    down_ref[...] += jnp.sum(e, axis=0, keepdims=True)

    @pl.when(i == nt - 1)
    def _fin():
        ftn = ft_scr[...]
        fb = ftn.astype(jnp.bfloat16).astype(jnp.float32)
        rb = lrowst_ref[...].astype(jnp.bfloat16).astype(jnp.float32)
        gl = jnp.sum(fb * rb, axis=0, keepdims=True)
        ups = jnp.exp2(gl * (_TEMP_INV * _LOG2E))
        d = down_ref[...]
        down = (d[:, 0:_B] + d[:, _B:2 * _B]
                + d[:, 2 * _B:3 * _B] + d[:, 3 * _B:4 * _B])
        out_ref[...] = ups / down


@functools.partial(jax.jit, static_argnames=())
def kernel(features, labels, centrals):
    ft = features.T
    c4 = centrals.reshape(_M // 4, 4 * _D)
    rows4 = _sc_gather(labels >> 2, c4)
    sub = rows4.reshape(_B, 4, _D)
    sel = (labels & 3)[:, None]
    lrows = jnp.where(
        sel == 0,
        sub[:, 0],
        jnp.where(sel == 1, sub[:, 1], jnp.where(sel == 2, sub[:, 2], sub[:, 3])),
    )
    lrowst = lrows.T
    nt = _M // (4 * _TMB)
    out = pl.pallas_call(
        _fused_kernel,
        grid=(nt,),
        in_specs=[
            pl.BlockSpec((_D, _B), lambda i: (0, 0)),
            pl.BlockSpec((_TMB, 4 * _D), lambda i: (i, 0)),
            pl.BlockSpec((_D, _B), lambda i: (0, 0)),
        ],
        out_specs=pl.BlockSpec((1, _B), lambda i: (0, 0)),
        out_shape=jax.ShapeDtypeStruct((1, _B), jnp.float32),
        scratch_shapes=[
            pltpu.VMEM((_D, _B), jnp.float32),
            pltpu.VMEM((4 * _D, 4 * _B), jnp.float32),
            pltpu.VMEM((1, 4 * _B), jnp.float32),
        ],
    )(ft, c4, lrowst)
    return out.reshape(_B)


# speedup vs baseline: 1.3778x; 1.2858x over previous
import functools

import jax
import jax.numpy as jnp
from jax import lax
from jax.experimental import pallas as pl
from jax.experimental.pallas import tpu as pltpu
from jax.experimental.pallas import tpu_sc as plsc

_TEMP_INV = 10.0
_LOG2E = 1.4426950408889634
_B = 1024
_D = 32
_M = 100000
_TM = 10000

_SC_INFO = plsc.get_sparse_core_info()
_NW = _SC_INFO.num_cores * _SC_INFO.num_subcores
_BPW = _B // _NW


def _sc_gather_body(idx_hbm, table_hbm, out_hbm, idx_v, rows_v, sem):
    wid = lax.axis_index("s") * _SC_INFO.num_cores + lax.axis_index("c")
    base = wid * _BPW
    pltpu.sync_copy(idx_hbm.at[pl.ds(base, _BPW)], idx_v)
    copies = []
    for h in range(_BPW // 16):
        v = idx_v[pl.ds(h * 16, 16)]
        for r in range(16):
            copies.append(
                pltpu.async_copy(
                    table_hbm.at[pl.ds(v[r], 1)],
                    rows_v.at[pl.ds(h * 16 + r, 1)],
                    sem,
                )
            )
    for c in copies:
        c.wait()
    pltpu.sync_copy(rows_v, out_hbm.at[pl.ds(base, _BPW)])


_sc_gather = functools.partial(
    pl.kernel,
    mesh=plsc.VectorSubcoreMesh(core_axis_name="c", subcore_axis_name="s"),
    out_type=jax.ShapeDtypeStruct((_B, _D), jnp.float32),
    scratch_types=[
        pltpu.VMEM((_BPW,), jnp.int32),
        pltpu.VMEM((_BPW, _D), jnp.float32),
        pltpu.SemaphoreType.DMA,
    ],
)(_sc_gather_body)


def _down_kernel(ft_ref, c_ref, down_ref, ft_scr, acc_ref):
    i = pl.program_id(0)
    nt = pl.num_programs(0)

    @pl.when(i == 0)
    def _init():
        ft = ft_ref[...]
        nrm = jnp.sqrt(jnp.sum(ft * ft, axis=0, keepdims=True))
        ft_scr[...] = ft / jnp.maximum(nrm, 1e-12)
        acc_ref[...] = jnp.zeros_like(acc_ref)

    ft = ft_scr[...]
    c = c_ref[...]
    g = jax.lax.dot_general(
        c, ft, (((1,), (0,)), ((), ())), preferred_element_type=jnp.float32
    )
    e = jnp.exp2(g * (_TEMP_INV * _LOG2E))
    acc_ref[...] += jnp.sum(e, axis=0, keepdims=True)

    @pl.when(i == nt - 1)
    def _fin():
        down_ref[...] = acc_ref[...]


def _finish_kernel(ft_ref, rows_ref, down_ref, out_ref):
    ft = ft_ref[...]
    nrm = jnp.sqrt(jnp.sum(ft * ft, axis=0, keepdims=True))
    ftn = ft / jnp.maximum(nrm, 1e-12)
    rt = jnp.transpose(rows_ref[...], (1, 0))
    fb = ftn.astype(jnp.bfloat16).astype(jnp.float32)
    rb = rt.astype(jnp.bfloat16).astype(jnp.float32)
    gl = jnp.sum(fb * rb, axis=0, keepdims=True)
    ups = jnp.exp2(gl * (_TEMP_INV * _LOG2E))
    out_ref[...] = ups / down_ref[...]


@functools.partial(jax.jit, static_argnames=())
def kernel(features, labels, centrals):
    ft = features.T
    rows = _sc_gather(labels, centrals)
    nt = _M // _TM
    down = pl.pallas_call(
        _down_kernel,
        grid=(nt,),
        in_specs=[
            pl.BlockSpec((_D, _B), lambda i: (0, 0)),
            pl.BlockSpec((_TM, _D), lambda i: (i, 0)),
        ],
        out_specs=pl.BlockSpec((1, _B), lambda i: (0, 0)),
        out_shape=jax.ShapeDtypeStruct((1, _B), jnp.float32),
        scratch_shapes=[
            pltpu.VMEM((_D, _B), jnp.float32),
            pltpu.VMEM((1, _B), jnp.float32),
        ],
    )(ft, centrals)
    out = pl.pallas_call(
        _finish_kernel,
        out_shape=jax.ShapeDtypeStruct((1, _B), jnp.float32),
    )(ft, rows, down)
    return out.reshape(_B)
